# baseline (speedup 1.0000x reference)
import jax
import jax.numpy as jnp
from jax.experimental import pallas as pl
from jax.experimental.pallas import tpu as pltpu

B, N, D, C = 8, 1024, 128, 128
BN = 1024
SUB = 4
BS = BN // SUB


def _body(a_ref, x_ref, w_ref, b_ref, o_ref, xwb_ref):
    xwb_ref[...] = (
        jnp.dot(x_ref[0], w_ref[...], preferred_element_type=jnp.float32)
        + b_ref[...]
    )
    xwb = xwb_ref[...]
    for t in range(SUB):
        a = a_ref[0, t * BS:(t + 1) * BS, :]
        u = jnp.dot(a, xwb, preferred_element_type=jnp.float32)
        ss = jnp.sum(u * u, axis=1, keepdims=True)
        out = u * jax.lax.rsqrt(jnp.maximum(ss, 1e-24))
        s = jnp.maximum(out, 0.0)
        e = jnp.exp(s)
        o_ref[0, t * BS:(t + 1) * BS, :] = e / jnp.sum(e, axis=1, keepdims=True)


@jax.jit
def kernel(input_tensor, tilda_adjacency_matrix, W, b):
    bias = b.reshape(1, C)
    grid = (B, N // BN)
    return pl.pallas_call(
        _body,
        grid=grid,
        in_specs=[
            pl.BlockSpec((1, BN, N), lambda bi, i: (bi, i, 0)),
            pl.BlockSpec((1, N, D), lambda bi, i: (bi, 0, 0)),
            pl.BlockSpec((D, C), lambda bi, i: (0, 0)),
            pl.BlockSpec((1, C), lambda bi, i: (0, 0)),
        ],
        out_specs=pl.BlockSpec((1, BN, C), lambda bi, i: (bi, i, 0)),
        out_shape=jax.ShapeDtypeStruct((B, N, C), jnp.float32),
        scratch_shapes=[pltpu.VMEM((N, C), jnp.float32)],
        compiler_params=pltpu.CompilerParams(
            dimension_semantics=("parallel", "arbitrary"),
        ),
    )(tilda_adjacency_matrix, input_tensor, W, bias)
